# Initial kernel scaffold; baseline (speedup 1.0000x reference)
import jax
import jax.numpy as jnp
from jax.experimental import pallas as pl


def kernel(hidden_states, positions, residual, ln1_w, ln2_w, W_qkv, b_qkv, W_o, b_o, sinks, W_router, b_router, w13, b13, w2, b2):
    raise NotImplementedError("write your pallas kernel here")


# speedup vs baseline: 3.2752x; 3.2752x over previous
import functools
import math

import jax
import jax.numpy as jnp
from jax.experimental import pallas as pl
from jax.experimental.pallas import tpu as pltpu

T = 2048
D = 2048
H = 16
KV = 4
DH = 128
E = 8
DFF = 1024
TOPK = 2
THETA = 150000.0
EPS = 1e-05
ALPHA = 1.702
LIMIT = 7.0
HALF = DH // 2

_F32 = jnp.float32


def _prologue_body(h_ref, r_ref, w_ref, res_ref, x_ref):
    res = h_ref[...] + r_ref[...]
    var = jnp.mean(res * res, axis=-1, keepdims=True)
    res_ref[...] = res
    x_ref[...] = res * jax.lax.rsqrt(var + EPS) * w_ref[...]


def _prologue(h, r, w):
    BM = 256
    return pl.pallas_call(
        _prologue_body,
        grid=(T // BM,),
        in_specs=[
            pl.BlockSpec((BM, D), lambda i: (i, 0)),
            pl.BlockSpec((BM, D), lambda i: (i, 0)),
            pl.BlockSpec((1, D), lambda i: (0, 0)),
        ],
        out_specs=[
            pl.BlockSpec((BM, D), lambda i: (i, 0)),
            pl.BlockSpec((BM, D), lambda i: (i, 0)),
        ],
        out_shape=[
            jax.ShapeDtypeStruct((T, D), _F32),
            jax.ShapeDtypeStruct((T, D), _F32),
        ],
    )(h, r, w.reshape(1, D))


def _matmul_bias_body(x_ref, w_ref, b_ref, o_ref):
    o_ref[...] = (
        jnp.dot(x_ref[...], w_ref[...], preferred_element_type=_F32)
        + b_ref[...]
    )


def _qkv_proj(x, w, b):
    N = (H + 2 * KV) * DH
    BM = 1024
    BN = 256
    return pl.pallas_call(
        _matmul_bias_body,
        grid=(T // BM, N // BN),
        in_specs=[
            pl.BlockSpec((BM, D), lambda i, n: (i, 0)),
            pl.BlockSpec((D, BN), lambda i, n: (0, n)),
            pl.BlockSpec((1, BN), lambda i, n: (0, n)),
        ],
        out_specs=pl.BlockSpec((BM, BN), lambda i, n: (i, n)),
        out_shape=jax.ShapeDtypeStruct((T, N), _F32),
    )(x, w, b.reshape(1, N))


def _rope_apply(x, c, s):
    x1 = x[:, :HALF]
    x2 = x[:, HALF:]
    return jnp.concatenate([x1 * c - x2 * s, x2 * c + x1 * s], axis=-1)


def _attn_body(q_ref, k_ref, v_ref, cq_ref, sq_ref, ck_ref, sk_ref,
               sinks_ref, o_ref, *, bq):
    h = pl.program_id(0)
    i = pl.program_id(1)
    qr = _rope_apply(q_ref[...], cq_ref[...], sq_ref[...])
    kr = _rope_apply(k_ref[...], ck_ref[...], sk_ref[...])
    scale = DH ** -0.5
    logits = jax.lax.dot_general(
        qr, kr, (((1,), (1,)), ((), ())), preferred_element_type=_F32
    ) * scale
    row = jax.lax.broadcasted_iota(jnp.int32, (bq, T), 0) + i * bq
    col = jax.lax.broadcasted_iota(jnp.int32, (bq, T), 1)
    logits = jnp.where(col <= row, logits, -1e30)
    sink = sinks_ref[h]
    m = jnp.maximum(jnp.max(logits, axis=-1, keepdims=True), sink)
    p = jnp.exp(logits - m)
    denom = jnp.sum(p, axis=-1, keepdims=True) + jnp.exp(sink - m)
    attn = p / denom
    o_ref[...] = jnp.dot(attn, v_ref[...], preferred_element_type=_F32)


def _attention(qkv, cos_t, sin_t, sinks):
    BQ = 256
    body = functools.partial(_attn_body, bq=BQ)
    return pl.pallas_call(
        body,
        grid=(H, T // BQ),
        in_specs=[
            pl.BlockSpec((BQ, DH), lambda h, i: (i, h)),
            pl.BlockSpec((T, DH), lambda h, i: (0, H + h // (H // KV))),
            pl.BlockSpec((T, DH), lambda h, i: (0, H + KV + h // (H // KV))),
            pl.BlockSpec((BQ, HALF), lambda h, i: (i, 0)),
            pl.BlockSpec((BQ, HALF), lambda h, i: (i, 0)),
            pl.BlockSpec((T, HALF), lambda h, i: (0, 0)),
            pl.BlockSpec((T, HALF), lambda h, i: (0, 0)),
            pl.BlockSpec(memory_space=pltpu.SMEM),
        ],
        out_specs=pl.BlockSpec((BQ, DH), lambda h, i: (i, h)),
        out_shape=jax.ShapeDtypeStruct((T, H * DH), _F32),
    )(qkv, qkv, qkv, cos_t, sin_t, cos_t, sin_t, sinks)


def _oproj_body(o_ref, w_ref, b_ref, res_ref, ln_ref, res_out_ref, x_ref,
                *, nk):
    k = pl.program_id(1)
    part = jnp.dot(o_ref[...], w_ref[...], preferred_element_type=_F32)

    @pl.when(k == 0)
    def _():
        res_out_ref[...] = part

    @pl.when(k > 0)
    def _():
        res_out_ref[...] += part

    @pl.when(k == nk - 1)
    def _():
        acc = res_out_ref[...] + b_ref[...] + res_ref[...]
        var = jnp.mean(acc * acc, axis=-1, keepdims=True)
        res_out_ref[...] = acc
        x_ref[...] = acc * jax.lax.rsqrt(var + EPS) * ln_ref[...]


def _oproj_norm(o, w, b, res, ln2_w):
    BM = 512
    BK = 512
    NK = (H * DH) // BK
    return pl.pallas_call(
        functools.partial(_oproj_body, nk=NK),
        grid=(T // BM, NK),
        in_specs=[
            pl.BlockSpec((BM, BK), lambda i, k: (i, k)),
            pl.BlockSpec((BK, D), lambda i, k: (k, 0)),
            pl.BlockSpec((1, D), lambda i, k: (0, 0)),
            pl.BlockSpec((BM, D), lambda i, k: (i, 0)),
            pl.BlockSpec((1, D), lambda i, k: (0, 0)),
        ],
        out_specs=[
            pl.BlockSpec((BM, D), lambda i, k: (i, 0)),
            pl.BlockSpec((BM, D), lambda i, k: (i, 0)),
        ],
        out_shape=[
            jax.ShapeDtypeStruct((T, D), _F32),
            jax.ShapeDtypeStruct((T, D), _F32),
        ],
    )(o, w, b.reshape(1, D), res, ln2_w.reshape(1, D))


def _router_body(x_ref, w_ref, b_ref, route_ref):
    g = jnp.dot(x_ref[...], w_ref[...], preferred_element_type=_F32) + b_ref[...]
    bm = g.shape[0]
    lane = jax.lax.broadcasted_iota(jnp.int32, (bm, 128), 1)
    valid = lane < E
    gm = jnp.where(valid, g, -1e30)
    m1 = jnp.max(gm, axis=-1, keepdims=True)
    idx1 = jnp.min(jnp.where(gm == m1, lane, 127), axis=-1, keepdims=True)
    oh1 = lane == idx1
    gm2 = jnp.where(oh1, -1e30, gm)
    m2 = jnp.max(gm2, axis=-1, keepdims=True)
    idx2 = jnp.min(jnp.where(gm2 == m2, lane, 127), axis=-1, keepdims=True)
    oh2 = lane == idx2
    w1 = 1.0 / (1.0 + jnp.exp(m2 - m1))
    w2 = 1.0 - w1
    route_ref[...] = jnp.where(oh1, w1, 0.0) + jnp.where(oh2, w2, 0.0)


def _router(x, w_router, b_router):
    BM = 256
    wp = jnp.zeros((D, 128), _F32).at[:, :E].set(w_router)
    bp = jnp.zeros((1, 128), _F32).at[0, :E].set(b_router)
    return pl.pallas_call(
        _router_body,
        grid=(T // BM,),
        in_specs=[
            pl.BlockSpec((BM, D), lambda i: (i, 0)),
            pl.BlockSpec((D, 128), lambda i: (0, 0)),
            pl.BlockSpec((1, 128), lambda i: (0, 0)),
        ],
        out_specs=pl.BlockSpec((BM, 128), lambda i: (i, 0)),
        out_shape=jax.ShapeDtypeStruct((T, 128), _F32),
    )(x, wp, bp)


def _moe_body(x_ref, wg_ref, wu_ref, w2_ref, bg_ref, bu_ref, b2_ref,
              route_ref, o_ref):
    e = pl.program_id(1)
    f = pl.program_id(2)
    x = x_ref[...]
    hg = jax.lax.dot_general(
        x, wg_ref[0], (((1,), (1,)), ((), ())), preferred_element_type=_F32
    ) + bg_ref[0]
    hu = jax.lax.dot_general(
        x, wu_ref[0], (((1,), (1,)), ((), ())), preferred_element_type=_F32
    ) + bu_ref[0]
    gate = jnp.minimum(hg, LIMIT)
    up = jnp.clip(hu, -LIMIT, LIMIT)
    act = gate * jax.nn.sigmoid(ALPHA * gate) * (up + 1.0)
    eo = jax.lax.dot_general(
        act, w2_ref[0], (((1,), (1,)), ((), ())),
        preferred_element_type=_F32
    )
    bm = x.shape[0]
    lane = jax.lax.broadcasted_iota(jnp.int32, (bm, 128), 1)
    w_e = jnp.sum(jnp.where(lane == e, route_ref[...], 0.0), axis=-1,
                  keepdims=True)
    contrib = w_e * eo

    @pl.when(f == 0)
    def _():
        contrib2 = contrib + w_e * b2_ref[0]

        @pl.when(e == 0)
        def _():
            o_ref[...] = contrib2

        @pl.when(e > 0)
        def _():
            o_ref[...] += contrib2

    @pl.when(f > 0)
    def _():
        o_ref[...] += contrib


def _moe(x, route, w13g, w13u, b13g, b13u, w2, b2):
    BM = 256
    BF = 512
    return pl.pallas_call(
        _moe_body,
        grid=(T // BM, E, DFF // BF),
        in_specs=[
            pl.BlockSpec((BM, D), lambda i, e, f: (i, 0)),
            pl.BlockSpec((1, BF, D), lambda i, e, f: (e, f, 0)),
            pl.BlockSpec((1, BF, D), lambda i, e, f: (e, f, 0)),
            pl.BlockSpec((1, D, BF), lambda i, e, f: (e, 0, f)),
            pl.BlockSpec((1, 1, BF), lambda i, e, f: (e, 0, f)),
            pl.BlockSpec((1, 1, BF), lambda i, e, f: (e, 0, f)),
            pl.BlockSpec((1, 1, D), lambda i, e, f: (e, 0, 0)),
            pl.BlockSpec((BM, 128), lambda i, e, f: (i, 0)),
        ],
        out_specs=pl.BlockSpec((BM, D), lambda i, e, f: (i, 0)),
        out_shape=jax.ShapeDtypeStruct((T, D), _F32),
    )(x, w13g, w13u, w2, b13g.reshape(E, 1, DFF), b13u.reshape(E, 1, DFF),
      b2.reshape(E, 1, D), route)


def kernel(hidden_states, positions, residual, ln1_w, ln2_w, W_qkv, b_qkv,
           W_o, b_o, sinks, W_router, b_router, w13, b13, w2, b2):
    res1, x = _prologue(hidden_states, residual, ln1_w)
    qkv = _qkv_proj(x, W_qkv, b_qkv)

    inv_freq = 1.0 / (THETA ** (jnp.arange(0, HALF, dtype=_F32) / HALF))
    ang = positions.astype(_F32)[:, None] * inv_freq[None, :]
    cos_t = jnp.cos(ang)
    sin_t = jnp.sin(ang)

    o = _attention(qkv, cos_t, sin_t, sinks)
    res2, x2 = _oproj_norm(o, W_o, b_o, res1, ln2_w)
    route = _router(x2, W_router, b_router)

    w13g = w13[:, 0::2, :]
    w13u = w13[:, 1::2, :]
    b13g = b13[:, 0::2]
    b13u = b13[:, 1::2]
    out = _moe(x2, route, w13g, w13u, b13g, b13u, w2, b2)
    return (out, res2)
